# Initial kernel scaffold; baseline (speedup 1.0000x reference)
import jax
import jax.numpy as jnp
from jax.experimental import pallas as pl


def kernel(x, table):
    raise NotImplementedError("write your pallas kernel here")


# speedup vs baseline: 2.9698x; 2.9698x over previous
import functools

import jax
import jax.numpy as jnp
from jax import lax
from jax.experimental import pallas as pl
from jax.experimental.pallas import tpu as pltpu
from jax.experimental.pallas import tpu_sc as plsc

_VOCAB = 100000
_D = 128
_B = 4096 * 50
_NW = 32
_BPW = _B // _NW
_C = 128
_NCHUNK = _BPW // _C


def _make_sc_gather():
    mesh = plsc.VectorSubcoreMesh(core_axis_name="c", subcore_axis_name="s")

    @functools.partial(
        pl.kernel,
        mesh=mesh,
        out_type=jax.ShapeDtypeStruct((_B, _D), jnp.float32),
        scratch_types=[
            pltpu.VMEM((_BPW,), jnp.int32),
            pltpu.VMEM((_C, _D), jnp.float32),
            pltpu.SemaphoreType.DMA,
        ],
    )
    def sc_gather(idx_hbm, table_hbm, out_hbm, idx_v, rows_v, sem0):
        wid = lax.axis_index("s") * 2 + lax.axis_index("c")
        base = wid * _BPW
        pltpu.sync_copy(idx_hbm.at[pl.ds(base, _BPW)], idx_v)

        def body(g, carry):
            pltpu.async_copy(
                table_hbm.at[idx_v.at[pl.ds(g * _C, _C)]],
                rows_v, sem0).wait()
            pltpu.sync_copy(rows_v, out_hbm.at[pl.ds(base + g * _C, _C)])
            return carry

        lax.fori_loop(0, _NCHUNK, body, 0)

    return sc_gather


_sc_gather = _make_sc_gather()


def kernel(x, table):
    flat_idx = x.reshape(-1).astype(jnp.int32)
    out = _sc_gather(flat_idx, table)
    return out.reshape(x.shape + (_D,))


# speedup vs baseline: 3.3310x; 1.1216x over previous
import functools

import jax
import jax.numpy as jnp
from jax import lax
from jax.experimental import pallas as pl
from jax.experimental.pallas import tpu as pltpu
from jax.experimental.pallas import tpu_sc as plsc

_VOCAB = 100000
_D = 128
_B = 4096 * 50
_NW = 32
_BPW = _B // _NW
_C = 128
_NCHUNK = _BPW // _C


def _make_sc_gather():
    mesh = plsc.VectorSubcoreMesh(core_axis_name="c", subcore_axis_name="s")

    @functools.partial(
        pl.kernel,
        mesh=mesh,
        out_type=jax.ShapeDtypeStruct((_B, _D), jnp.float32),
        scratch_types=[
            pltpu.VMEM((_BPW,), jnp.int32),
            pltpu.VMEM((_C, _D), jnp.float32),
            pltpu.VMEM((_C, _D), jnp.float32),
            pltpu.SemaphoreType.DMA,
            pltpu.SemaphoreType.DMA,
        ],
    )
    def sc_gather(idx_hbm, table_hbm, out_hbm, idx_v, buf0, buf1, sem0, sem1):
        wid = lax.axis_index("s") * 2 + lax.axis_index("c")
        base = wid * _BPW
        pltpu.sync_copy(idx_hbm.at[pl.ds(base, _BPW)], idx_v)

        def gather(g, buf, sem):
            pltpu.async_copy(
                table_hbm.at[idx_v.at[pl.ds(g * _C, _C)]], buf, sem)

        def drain(g, buf, sem):
            pltpu.make_async_copy(
                table_hbm.at[idx_v.at[pl.ds(g * _C, _C)]], buf, sem).wait()
            pltpu.sync_copy(buf, out_hbm.at[pl.ds(base + g * _C, _C)])

        gather(0, buf0, sem0)

        def body(t, carry):
            g0 = 2 * t
            gather(g0 + 1, buf1, sem1)
            drain(g0, buf0, sem0)
            gather(g0 + 2, buf0, sem0)
            drain(g0 + 1, buf1, sem1)
            return carry

        lax.fori_loop(0, _NCHUNK // 2 - 1, body, 0)
        gather(_NCHUNK - 1, buf1, sem1)
        drain(_NCHUNK - 2, buf0, sem0)
        drain(_NCHUNK - 1, buf1, sem1)

    return sc_gather


_sc_gather = _make_sc_gather()


def kernel(x, table):
    flat_idx = x.reshape(-1).astype(jnp.int32)
    out = _sc_gather(flat_idx, table)
    return out.reshape(x.shape + (_D,))


# speedup vs baseline: 3.3349x; 1.0012x over previous
import functools

import jax
import jax.numpy as jnp
from jax import lax
from jax.experimental import pallas as pl
from jax.experimental.pallas import tpu as pltpu
from jax.experimental.pallas import tpu_sc as plsc

_VOCAB = 100000
_D = 128
_B = 4096 * 50
_NW = 32
_BPW = _B // _NW
_C = 128
_NCHUNK = _BPW // _C


def _make_sc_gather():
    mesh = plsc.VectorSubcoreMesh(core_axis_name="c", subcore_axis_name="s")

    @functools.partial(
        pl.kernel,
        mesh=mesh,
        out_type=jax.ShapeDtypeStruct((_B, _D), jnp.float32),
        scratch_types=[
            pltpu.VMEM((_BPW,), jnp.int32),
            pltpu.VMEM((_C, _D), jnp.float32),
            pltpu.VMEM((_C, _D), jnp.float32),
            pltpu.VMEM((_C, _D), jnp.float32),
            pltpu.VMEM((_C, _D), jnp.float32),
            pltpu.SemaphoreType.DMA,
            pltpu.SemaphoreType.DMA,
            pltpu.SemaphoreType.DMA,
            pltpu.SemaphoreType.DMA,
            pltpu.SemaphoreType.DMA,
            pltpu.SemaphoreType.DMA,
            pltpu.SemaphoreType.DMA,
            pltpu.SemaphoreType.DMA,
        ],
    )
    def sc_gather(idx_hbm, table_hbm, out_hbm, idx_v,
                  b0, b1, b2, b3, g0s, g1s, g2s, g3s, s0, s1, s2, s3):
        bufs = [b0, b1, b2, b3]
        gsem = [g0s, g1s, g2s, g3s]
        ssem = [s0, s1, s2, s3]
        wid = lax.axis_index("s") * 2 + lax.axis_index("c")
        base = wid * _BPW
        pltpu.sync_copy(idx_hbm.at[pl.ds(base, _BPW)], idx_v)

        def gather(g, b):
            pltpu.async_copy(
                table_hbm.at[idx_v.at[pl.ds(g * _C, _C)]], bufs[b], gsem[b])

        def gwait(g, b):
            pltpu.make_async_copy(
                table_hbm.at[idx_v.at[pl.ds(g * _C, _C)]],
                bufs[b], gsem[b]).wait()

        def astore(g, b):
            pltpu.async_copy(
                bufs[b], out_hbm.at[pl.ds(base + g * _C, _C)], ssem[b])

        def swait(g, b):
            pltpu.make_async_copy(
                bufs[b], out_hbm.at[pl.ds(base + g * _C, _C)],
                ssem[b]).wait()

        gather(0, 0)
        gather(1, 1)
        gather(2, 2)
        gwait(0, 0)
        astore(0, 0)
        gather(3, 3)

        def body(t, carry):
            gbase = 4 * t + 1
            for k in range(4):
                g = gbase + k
                b = (1 + k) % 4
                gwait(g, b)
                astore(g, b)
                swait(g - 1, (b - 1) % 4)
                gather(g + 3, (b - 1) % 4)
            return carry

        lax.fori_loop(0, (_NCHUNK - 6) // 4, body, 0)

        ge = _NCHUNK - 5
        gwait(ge, 1)
        astore(ge, 1)
        swait(ge - 1, 0)
        gather(ge + 3, 0)
        gwait(ge + 1, 2)
        astore(ge + 1, 2)
        swait(ge, 1)
        gather(ge + 4, 1)
        gwait(ge + 2, 3)
        astore(ge + 2, 3)
        gwait(ge + 3, 0)
        astore(ge + 3, 0)
        gwait(ge + 4, 1)
        astore(ge + 4, 1)
        swait(ge + 1, 2)
        swait(ge + 2, 3)
        swait(ge + 3, 0)
        swait(ge + 4, 1)

    return sc_gather


_sc_gather = _make_sc_gather()


def kernel(x, table):
    flat_idx = x.reshape(-1).astype(jnp.int32)
    out = _sc_gather(flat_idx, table)
    return out.reshape(x.shape + (_D,))


# speedup vs baseline: 5.9555x; 1.7858x over previous
import functools

import jax
import jax.numpy as jnp
from jax import lax
from jax.experimental import pallas as pl
from jax.experimental.pallas import tpu as pltpu
from jax.experimental.pallas import tpu_sc as plsc

_VOCAB = 100000
_D = 128
_BATCH = 4096
_HIST = 50
_B = _BATCH * _HIST
_NW = 32
_BPW = _B // _NW
_EPW = _BATCH // _NW
_EPC = 4
_C = _EPC * _HIST
_NCHUNK = _BPW // _C


def _make_sc_gather():
    mesh = plsc.VectorSubcoreMesh(core_axis_name="c", subcore_axis_name="s")

    @functools.partial(
        pl.kernel,
        mesh=mesh,
        out_type=jax.ShapeDtypeStruct((_BATCH, _HIST, _D), jnp.float32),
        scratch_types=[
            pltpu.VMEM((_EPW, _HIST), jnp.int32),
            pltpu.VMEM((_EPC, _HIST, _D), jnp.float32),
            pltpu.VMEM((_EPC, _HIST, _D), jnp.float32),
            pltpu.VMEM((_EPC, _HIST, _D), jnp.float32),
            pltpu.VMEM((_EPC, _HIST, _D), jnp.float32),
            pltpu.SemaphoreType.DMA,
            pltpu.SemaphoreType.DMA,
            pltpu.SemaphoreType.DMA,
            pltpu.SemaphoreType.DMA,
            pltpu.SemaphoreType.DMA,
            pltpu.SemaphoreType.DMA,
            pltpu.SemaphoreType.DMA,
            pltpu.SemaphoreType.DMA,
        ],
    )
    def sc_gather(idx_hbm, table_hbm, out_hbm, idx_v,
                  b0, b1, b2, b3, g0s, g1s, g2s, g3s, s0, s1, s2, s3):
        bufs = [b0, b1, b2, b3]
        gsem = [g0s, g1s, g2s, g3s]
        ssem = [s0, s1, s2, s3]
        wid = lax.axis_index("s") * 2 + lax.axis_index("c")
        ebase = wid * _EPW
        pltpu.sync_copy(idx_hbm.at[pl.ds(ebase, _EPW)], idx_v)

        def gather(g, b):
            for j in range(_EPC):
                pltpu.async_copy(
                    table_hbm.at[idx_v.at[g * _EPC + j]],
                    bufs[b].at[j], gsem[b])

        def gwait(g, b):
            for j in range(_EPC):
                pltpu.make_async_copy(
                    table_hbm.at[idx_v.at[g * _EPC + j]],
                    bufs[b].at[j], gsem[b]).wait()

        def astore(g, b):
            pltpu.async_copy(
                bufs[b], out_hbm.at[pl.ds(ebase + g * _EPC, _EPC)], ssem[b])

        def swait(g, b):
            pltpu.make_async_copy(
                bufs[b], out_hbm.at[pl.ds(ebase + g * _EPC, _EPC)],
                ssem[b]).wait()

        gather(0, 0)
        gather(1, 1)
        gather(2, 2)
        gwait(0, 0)
        astore(0, 0)
        gather(3, 3)

        def body(t, carry):
            gbase = 4 * t + 1
            for k in range(4):
                g = gbase + k
                b = (1 + k) % 4
                gwait(g, b)
                astore(g, b)
                swait(g - 1, (b - 1) % 4)
                gather(g + 3, (b - 1) % 4)
            return carry

        T = (_NCHUNK - 4) // 4
        lax.fori_loop(0, T, body, 0)

        waited = 4 * T
        for g in range(4 * T + 1, _NCHUNK):
            gwait(g, g % 4)
            astore(g, g % 4)
            nxt = g + 3
            if nxt <= _NCHUNK - 1:
                swait(g - 1, (g - 1) % 4)
                waited = g
                gather(nxt, nxt % 4)
        for g in range(waited, _NCHUNK):
            swait(g, g % 4)

    return sc_gather


_sc_gather = _make_sc_gather()


def kernel(x, table):
    return _sc_gather(x.astype(jnp.int32), table)


# speedup vs baseline: 10.7032x; 1.7972x over previous
import functools

import jax
import jax.numpy as jnp
from jax import lax
from jax.experimental import pallas as pl
from jax.experimental.pallas import tpu as pltpu
from jax.experimental.pallas import tpu_sc as plsc

_VOCAB = 100000
_D = 128
_BATCH = 4096
_HIST = 50
_NW = 32
_EPW = _BATCH // _NW
_NCHUNK = _HIST


def _make_sc_gather():
    mesh = plsc.VectorSubcoreMesh(core_axis_name="c", subcore_axis_name="s")

    @functools.partial(
        pl.kernel,
        mesh=mesh,
        out_type=jax.ShapeDtypeStruct((_HIST, _BATCH, _D), jnp.float32),
        scratch_types=[
            pltpu.VMEM((_HIST, _EPW), jnp.int32),
            pltpu.VMEM((_EPW, _D), jnp.float32),
            pltpu.VMEM((_EPW, _D), jnp.float32),
            pltpu.VMEM((_EPW, _D), jnp.float32),
            pltpu.VMEM((_EPW, _D), jnp.float32),
            pltpu.SemaphoreType.DMA,
            pltpu.SemaphoreType.DMA,
            pltpu.SemaphoreType.DMA,
            pltpu.SemaphoreType.DMA,
            pltpu.SemaphoreType.DMA,
            pltpu.SemaphoreType.DMA,
            pltpu.SemaphoreType.DMA,
            pltpu.SemaphoreType.DMA,
        ],
    )
    def sc_gather(idx_hbm, table_hbm, out_hbm, idx_v,
                  b0, b1, b2, b3, g0s, g1s, g2s, g3s, s0, s1, s2, s3):
        bufs = [b0, b1, b2, b3]
        gsem = [g0s, g1s, g2s, g3s]
        ssem = [s0, s1, s2, s3]
        wid = lax.axis_index("s") * 2 + lax.axis_index("c")
        ebase = wid * _EPW
        pltpu.sync_copy(idx_hbm.at[:, pl.ds(ebase, _EPW)], idx_v)

        def gather(g, b):
            pltpu.async_copy(
                table_hbm.at[idx_v.at[g]], bufs[b], gsem[b])

        def gwait(g, b):
            pltpu.make_async_copy(
                table_hbm.at[idx_v.at[g]], bufs[b], gsem[b]).wait()

        def astore(g, b):
            pltpu.async_copy(
                bufs[b], out_hbm.at[g, pl.ds(ebase, _EPW)], ssem[b])

        def swait(g, b):
            pltpu.make_async_copy(
                bufs[b], out_hbm.at[g, pl.ds(ebase, _EPW)], ssem[b]).wait()

        gather(0, 0)
        gather(1, 1)
        gather(2, 2)
        gwait(0, 0)
        astore(0, 0)
        gather(3, 3)

        def body(t, carry):
            gbase = 4 * t + 1
            for k in range(4):
                g = gbase + k
                b = (1 + k) % 4
                gwait(g, b)
                astore(g, b)
                swait(g - 1, (b - 1) % 4)
                gather(g + 3, (b - 1) % 4)
            return carry

        T = (_NCHUNK - 4) // 4
        lax.fori_loop(0, T, body, 0)

        waited = 4 * T
        for g in range(4 * T + 1, _NCHUNK):
            gwait(g, g % 4)
            astore(g, g % 4)
            nxt = g + 3
            if nxt <= _NCHUNK - 1:
                swait(g - 1, (g - 1) % 4)
                waited = g
                gather(nxt, nxt % 4)
        for g in range(waited, _NCHUNK):
            swait(g, g % 4)

    return sc_gather


_sc_gather = _make_sc_gather()


def kernel(x, table):
    xt = x.T.astype(jnp.int32)
    out = _sc_gather(xt, table)
    return jnp.transpose(out, (1, 0, 2))
